# Initial kernel scaffold; baseline (speedup 1.0000x reference)
import jax
import jax.numpy as jnp
from jax.experimental import pallas as pl


def kernel(rpn_prob, rpn_roi):
    raise NotImplementedError("write your pallas kernel here")


# speedup vs baseline: 153.7040x; 153.7040x over previous
import jax
import jax.numpy as jnp
from jax import lax
from jax.experimental import pallas as pl
from jax.experimental.pallas import tpu as pltpu

_N_SCORE = 5000
_N_NMS = 1000
_TH = 0.7
_TILE = 128
_NT = 40
_NPAD = _NT * _TILE
_OUT_PAD = 1024


def _nms_body(boxes_ref, boxesT_ref, out_ref, keep_ref):
    out_ref[0] = jnp.zeros((_OUT_PAD, 4), jnp.float32)

    iota_r = lax.broadcasted_iota(jnp.int32, (_TILE, _TILE), 0)
    iota_c = lax.broadcasted_iota(jnp.int32, (_TILE, _TILE), 1)
    lower_tri = jnp.where(iota_r > iota_c, 1.0, 0.0)
    incl_tri = jnp.where(iota_r <= iota_c, 1.0, 0.0)
    eye = jnp.where(iota_r == iota_c, 1.0, 0.0)
    col_iota = lax.broadcasted_iota(jnp.int32, (_TILE, 1), 0)
    out_iota = lax.broadcasted_iota(jnp.int32, (_OUT_PAD, 1), 0)

    def iou_vs(y1r, x1r, y2r, x2r, area_r, tile_t):
        y1c = tile_t[0:1, :]
        x1c = tile_t[1:2, :]
        y2c = tile_t[2:3, :]
        x2c = tile_t[3:4, :]
        area_c = (y2c - y1c) * (x2c - x1c)
        ih = jnp.maximum(0.0, jnp.minimum(y2r, y2c) - jnp.maximum(y1r, y1c))
        iw = jnp.maximum(0.0, jnp.minimum(x2r, x2c) - jnp.maximum(x1r, x1c))
        inter = ih * iw
        union = area_r + area_c - inter
        return jnp.where(union > 0.0, inter / union, 0.0)

    def process_tile(carry):
        i, cnt = carry
        tile = boxes_ref[0, i]
        tile_t = boxesT_ref[0, i]
        y1r = tile[:, 0:1]
        x1r = tile[:, 1:2]
        y2r = tile[:, 2:3]
        x2r = tile[:, 3:4]
        area_r = (y2r - y1r) * (x2r - x1r)

        def cross(j, sup):
            iou = iou_vs(y1r, x1r, y2r, x2r, area_r, boxesT_ref[0, j])
            keep_j = keep_ref[j, 0:1, :]
            hit = jnp.where((iou > _TH) & (keep_j > 0.0), 1.0, 0.0)
            return sup + jnp.sum(hit, axis=1, keepdims=True)

        sup = lax.fori_loop(0, i, cross, jnp.zeros((_TILE, 1), jnp.float32))

        iou_self = iou_vs(y1r, x1r, y2r, x2r, area_r, tile_t)
        m_mat = jnp.where(iou_self > _TH, 1.0, 0.0) * lower_tri
        valid = jnp.where(i * _TILE + col_iota < _N_SCORE, 1.0, 0.0)
        base = valid * jnp.where(sup > 0.0, 0.0, 1.0)

        def fix_step(k):
            s = jnp.dot(m_mat, k, preferred_element_type=jnp.float32)
            return base * jnp.where(s > 0.0, 0.0, 1.0)

        def fix_cond(c):
            old, new = c
            return jnp.any(old != new)

        def fix_body(c):
            _, k = c
            return k, fix_step(k)

        _, keep = lax.while_loop(fix_cond, fix_body, (base, fix_step(base)))

        keep_row = jnp.sum(keep * eye, axis=0, keepdims=True)
        keep_ref[i, 0:1, :] = keep_row

        cum = jnp.dot(keep_row, incl_tri, preferred_element_type=jnp.float32)
        pos = cnt + cum.astype(jnp.int32) - 1
        onehot = jnp.where((out_iota == pos) & (keep_row > 0.0), 1.0, 0.0)
        cols = [
            jnp.sum(onehot * tile_t[c : c + 1, :], axis=1, keepdims=True)
            for c in range(4)
        ]
        out_ref[0] += jnp.concatenate(cols, axis=1)

        new_cnt = cnt + jnp.sum(keep_row).astype(jnp.int32)
        return i + 1, new_cnt

    def outer_cond(carry):
        i, cnt = carry
        return (i < _NT) & (cnt < _N_NMS)

    lax.while_loop(outer_cond, process_tile, (jnp.int32(0), jnp.int32(0)))


def _nms_call(boxes, boxes_t):
    b = boxes.shape[0]
    return pl.pallas_call(
        _nms_body,
        grid=(b,),
        in_specs=[
            pl.BlockSpec((1, _NT, _TILE, 4), lambda bb: (bb, 0, 0, 0)),
            pl.BlockSpec((1, _NT, 4, _TILE), lambda bb: (bb, 0, 0, 0)),
        ],
        out_specs=pl.BlockSpec((1, _OUT_PAD, 4), lambda bb: (bb, 0, 0)),
        out_shape=jax.ShapeDtypeStruct((b, _OUT_PAD, 4), jnp.float32),
        scratch_shapes=[pltpu.VMEM((_NT, 8, _TILE), jnp.float32)],
    )(boxes, boxes_t)


def kernel(rpn_prob, rpn_roi):
    b = rpn_prob.shape[0]
    scores = rpn_prob[..., 0]
    _, top_idx = lax.top_k(scores, _N_SCORE)
    top_roi = jnp.take_along_axis(rpn_roi, top_idx[..., None], axis=1)
    pad = jnp.zeros((b, _NPAD - _N_SCORE, 4), top_roi.dtype)
    boxes = jnp.concatenate([top_roi, pad], axis=1).reshape(b, _NT, _TILE, 4)
    boxes_t = jnp.swapaxes(boxes, 2, 3)
    out = _nms_call(boxes, boxes_t)
    return out[:, :_N_NMS, :]
